# baseline (speedup 1.0000x reference)
import functools

import jax
import jax.numpy as jnp
from jax import lax
from jax.experimental import pallas as pl
from jax.experimental.pallas import tpu as pltpu
from jax.experimental.pallas import tpu_sc as plsc

N = 10000
E = 320000
D = 128
NUM_RELS = 200

NC = 2
NS = 16
NW = NC * NS
CHUNK = 40
GIDX = 2 * CHUNK
NCHUNK = 256
EPW = NCHUNK * CHUNK
IDXC = 32
NSUP = NCHUNK // IDXC
RING = 4
LEAD = RING - 2
GRP = RING
NGRP = IDXC // GRP
EPAD = EPW * NW
NPAD = 10112
ROWS_PT = NPAD // NS
LANES = 16


def _sc_message_accumulate(gidx4, dst4, tab, zrows):
  mesh = plsc.VectorSubcoreMesh(core_axis_name="c", subcore_axis_name="s")

  @functools.partial(
      pl.kernel,
      out_type=jax.ShapeDtypeStruct((NC, NPAD, D), jnp.float32),
      mesh=mesh,
      scratch_types=[
          pltpu.VMEM((IDXC, GIDX), jnp.int32),
          pltpu.VMEM((IDXC, CHUNK), jnp.int32),
          [pltpu.VMEM((GIDX, D), jnp.float32) for _ in range(RING)],
          pltpu.VMEM_SHARED((NPAD, D), jnp.float32),
          [pltpu.SemaphoreType.DMA for _ in range(RING)],
          [pltpu.SemaphoreType.DMA for _ in range(RING)],
      ],
  )
  def k(gidx_hbm, dst_hbm, tab_hbm, z_hbm, out_hbm,
        gidx_v, dst_v, bufs, acc_sh, sem_g, sem_s):
    cid = lax.axis_index("c")
    sid = lax.axis_index("s")
    wid = cid * NS + sid

    pltpu.sync_copy(z_hbm, acc_sh.at[pl.ds(sid * ROWS_PT, ROWS_PT)])
    plsc.subcore_barrier()

    def issue_gather(c, r):
      pltpu.async_copy(tab_hbm.at[gidx_v.at[c]], bufs[r], sem_g[r])

    def wait_gather(c, r):
      pltpu.make_async_copy(tab_hbm.at[gidx_v.at[c]], bufs[r],
                            sem_g[r]).wait()

    def issue_scatter(c, r):
      pltpu.async_copy(bufs[r].at[pl.ds(0, CHUNK)],
                       acc_sh.at[dst_v.at[c]], sem_s[r], add=True)

    def wait_scatter(c, r):
      pltpu.make_async_copy(bufs[r].at[pl.ds(0, CHUNK)],
                            acc_sh.at[dst_v.at[c]], sem_s[r]).wait()

    def mul(r):
      buf = bufs[r]

      def mul_body(e, carry2):
        for j in range(D // LANES):
          s = pl.ds(j * LANES, LANES)
          buf[e, s] = buf[e, s] * buf[CHUNK + e, s]
        return carry2

      lax.fori_loop(0, CHUNK, mul_body, 0)

    def super_body(sc, carry):
      pltpu.sync_copy(gidx_hbm.at[wid, sc], gidx_v)
      pltpu.sync_copy(dst_hbm.at[wid, sc], dst_v)

      for c in range(LEAD):
        issue_gather(c, c % RING)

      def group_body(g, carry1):
        base = g * GRP
        for k in range(GRP):
          c = base + k
          if k >= LEAD:
            wait_scatter(c - LEAD, (k - LEAD) % RING)
          else:
            @pl.when(g >= 1)
            def _():
              wait_scatter(c - LEAD, (k - LEAD + GRP) % RING)
          if k < GRP - LEAD:
            issue_gather(c + LEAD, (k + LEAD) % RING)
          else:
            @pl.when(g + 1 < NGRP)
            def _():
              issue_gather(c + LEAD, (k + LEAD) % RING)
          wait_gather(c, k % RING)
          mul(k % RING)
          issue_scatter(c, k % RING)
        return carry1

      lax.fori_loop(0, NGRP, group_body, 0)
      for i in range(LEAD):
        c = IDXC - LEAD + i
        wait_scatter(c, c % RING)
      return carry

    lax.fori_loop(0, NSUP, super_body, 0)
    plsc.subcore_barrier()

    r0 = sid * ROWS_PT
    pltpu.sync_copy(acc_sh.at[pl.ds(r0, ROWS_PT)],
                    out_hbm.at[cid, pl.ds(r0, ROWS_PT)])

  return k(gidx4, dst4, tab, zrows)


def _tc_out_matmul(partials, weight, bias2d):
  BM = 1264

  def body(p_ref, w_ref, b_ref, o_ref):
    acc = jnp.dot(p_ref[0] + p_ref[1], w_ref[...],
                  preferred_element_type=jnp.float32)
    o_ref[...] = acc + b_ref[...]

  return pl.pallas_call(
      body,
      grid=(NPAD // BM,),
      in_specs=[
          pl.BlockSpec((NC, BM, D), lambda i: (0, i, 0)),
          pl.BlockSpec((D, D), lambda i: (0, 0)),
          pl.BlockSpec((1, D), lambda i: (0, 0)),
      ],
      out_specs=pl.BlockSpec((BM, D), lambda i: (i, 0)),
      out_shape=jax.ShapeDtypeStruct((NPAD, D), jnp.float32),
  )(partials, weight, bias2d)


def _tc_rel_matmul(re_pad, rel_weight):
  def body(r_ref, w_ref, o_ref):
    o_ref[...] = jnp.dot(r_ref[...], w_ref[...],
                         preferred_element_type=jnp.float32)

  return pl.pallas_call(
      body,
      out_shape=jax.ShapeDtypeStruct((re_pad.shape[0], D), jnp.float32),
  )(re_pad, rel_weight)


def kernel(x, edge_index, edge_type, rel_embed, weight, rel_weight, bias):
  src = edge_index[0]
  dst = edge_index[1]
  npad = EPAD - E
  spread = jnp.arange(npad, dtype=jnp.int32) % N
  src_p = jnp.concatenate([src, spread]).reshape(NW, NSUP, IDXC, CHUNK)
  et_p = jnp.concatenate(
      [edge_type, jnp.full((npad,), 2 * NUM_RELS, jnp.int32)]
  ).reshape(NW, NSUP, IDXC, CHUNK)
  gidx4 = jnp.concatenate([src_p, et_p + N], axis=3)
  dst4 = jnp.concatenate([dst, spread]).reshape(NW, NSUP, IDXC, CHUNK)

  re_ext = jnp.concatenate(
      [rel_embed, jnp.zeros((1, D), rel_embed.dtype)], axis=0)
  tab = jnp.concatenate([x, re_ext], axis=0)
  zrows = jnp.zeros((ROWS_PT, D), jnp.float32)

  partials = _sc_message_accumulate(gidx4, dst4, tab, zrows)
  out = _tc_out_matmul(partials, weight, bias.reshape(1, D))[:N]

  re_pad = jnp.concatenate(
      [re_ext, jnp.zeros((7, D), rel_embed.dtype)], axis=0)
  rel_out = _tc_rel_matmul(re_pad, rel_weight)[:2 * NUM_RELS + 1]
  return (out, rel_out)


# speedup vs baseline: 1.2835x; 1.2835x over previous
import functools

import jax
import jax.numpy as jnp
from jax import lax
from jax.experimental import pallas as pl
from jax.experimental.pallas import tpu as pltpu
from jax.experimental.pallas import tpu_sc as plsc

N = 10000
E = 320000
D = 128
NUM_RELS = 200

NC = 2
NS = 16
NW = NC * NS
CHUNK = 16
GIDX = 2 * CHUNK
NCHUNK = 640
EPW = NCHUNK * CHUNK
IDXC = 64
NSUP = NCHUNK // IDXC
RING = 8
LEAD = RING - 2
GRP = RING
NGRP = IDXC // GRP
EPAD = EPW * NW
NPAD = 10112
ROWS_PT = NPAD // NS
LANES = 16


def _sc_message_accumulate(gidx4, dst4, tab, zrows):
  mesh = plsc.VectorSubcoreMesh(core_axis_name="c", subcore_axis_name="s")

  @functools.partial(
      pl.kernel,
      out_type=jax.ShapeDtypeStruct((NC, NPAD, D), jnp.float32),
      mesh=mesh,
      compiler_params=pltpu.CompilerParams(use_tc_tiling_on_sc=False),
      scratch_types=[
          pltpu.VMEM((IDXC, GIDX), jnp.int32),
          pltpu.VMEM((IDXC, CHUNK), jnp.int32),
          [pltpu.VMEM((GIDX, D // 2), jnp.int32) for _ in range(RING)],
          [pltpu.VMEM((CHUNK, D), jnp.float32) for _ in range(RING)],
          pltpu.VMEM_SHARED((NPAD, D), jnp.float32),
          [pltpu.SemaphoreType.DMA for _ in range(RING)],
          [pltpu.SemaphoreType.DMA for _ in range(RING)],
      ],
  )
  def k(gidx_hbm, dst_hbm, tab_hbm, z_hbm, out_hbm,
        gidx_v, dst_v, bufs, prods, acc_sh, sem_g, sem_s):
    cid = lax.axis_index("c")
    sid = lax.axis_index("s")
    wid = cid * NS + sid

    pltpu.sync_copy(z_hbm, acc_sh.at[pl.ds(sid * ROWS_PT, ROWS_PT)])
    plsc.subcore_barrier()

    def issue_gather(c, r):
      pltpu.async_copy(tab_hbm.at[gidx_v.at[c]], bufs[r], sem_g[r])

    def wait_gather(c, r):
      pltpu.make_async_copy(tab_hbm.at[gidx_v.at[c]], bufs[r],
                            sem_g[r]).wait()

    def issue_scatter(c, r):
      pltpu.async_copy(prods[r], acc_sh.at[dst_v.at[c]], sem_s[r], add=True)

    def wait_scatter(c, r):
      pltpu.make_async_copy(prods[r], acc_sh.at[dst_v.at[c]],
                            sem_s[r]).wait()

    def mul(r):
      buf = bufs[r]
      prod = prods[r]

      hi_mask = jnp.int32(-65536)

      def mul_body(e, carry2):
        for j in range(D // (2 * LANES)):
          s = pl.ds(j * LANES, LANES)
          wx = buf[e, s]
          wr = buf[CHUNK + e, s]
          xa = lax.bitcast_convert_type(wx << 16, jnp.float32)
          xb = lax.bitcast_convert_type(wx & hi_mask, jnp.float32)
          ra = lax.bitcast_convert_type(wr << 16, jnp.float32)
          rb = lax.bitcast_convert_type(wr & hi_mask, jnp.float32)
          prod[e, pl.ds(j * 2 * LANES, LANES)] = xa * ra
          prod[e, pl.ds(j * 2 * LANES + LANES, LANES)] = xb * rb
        return carry2

      lax.fori_loop(0, CHUNK, mul_body, 0)

    def super_body(sc, carry):
      pltpu.sync_copy(gidx_hbm.at[wid, sc], gidx_v)
      pltpu.sync_copy(dst_hbm.at[wid, sc], dst_v)

      for c in range(LEAD):
        issue_gather(c, c % RING)

      def group_body(g, carry1):
        base = g * GRP
        for k in range(GRP):
          c = base + k
          if k >= LEAD:
            wait_scatter(c - LEAD, (k - LEAD) % RING)
          else:
            @pl.when(g >= 1)
            def _():
              wait_scatter(c - LEAD, (k - LEAD + GRP) % RING)
          if k < GRP - LEAD:
            issue_gather(c + LEAD, (k + LEAD) % RING)
          else:
            @pl.when(g + 1 < NGRP)
            def _():
              issue_gather(c + LEAD, (k + LEAD) % RING)
          wait_gather(c, k % RING)
          mul(k % RING)
          issue_scatter(c, k % RING)
        return carry1

      lax.fori_loop(0, NGRP, group_body, 0)
      for i in range(LEAD):
        c = IDXC - LEAD + i
        wait_scatter(c, c % RING)
      return carry

    lax.fori_loop(0, NSUP, super_body, 0)
    plsc.subcore_barrier()

    r0 = sid * ROWS_PT
    pltpu.sync_copy(acc_sh.at[pl.ds(r0, ROWS_PT)],
                    out_hbm.at[cid, pl.ds(r0, ROWS_PT)])

  return k(gidx4, dst4, tab, zrows)


def _tc_out_matmul(partials, weight, bias2d):
  BM = 1264

  def body(p_ref, w_ref, b_ref, o_ref):
    acc = jnp.dot(p_ref[0] + p_ref[1], w_ref[...],
                  preferred_element_type=jnp.float32)
    o_ref[...] = acc + b_ref[...]

  return pl.pallas_call(
      body,
      grid=(NPAD // BM,),
      in_specs=[
          pl.BlockSpec((NC, BM, D), lambda i: (0, i, 0)),
          pl.BlockSpec((D, D), lambda i: (0, 0)),
          pl.BlockSpec((1, D), lambda i: (0, 0)),
      ],
      out_specs=pl.BlockSpec((BM, D), lambda i: (i, 0)),
      out_shape=jax.ShapeDtypeStruct((NPAD, D), jnp.float32),
  )(partials, weight, bias2d)


def _tc_rel_matmul(re_pad, rel_weight):
  def body(r_ref, w_ref, o_ref):
    o_ref[...] = jnp.dot(r_ref[...], w_ref[...],
                         preferred_element_type=jnp.float32)

  return pl.pallas_call(
      body,
      out_shape=jax.ShapeDtypeStruct((re_pad.shape[0], D), jnp.float32),
  )(re_pad, rel_weight)


def kernel(x, edge_index, edge_type, rel_embed, weight, rel_weight, bias):
  src = edge_index[0]
  dst = edge_index[1]
  npad = EPAD - E
  spread = jnp.arange(npad, dtype=jnp.int32) % N
  src_p = jnp.concatenate([src, spread]).reshape(NW, NSUP, IDXC, CHUNK)
  et_p = jnp.concatenate(
      [edge_type, jnp.full((npad,), 2 * NUM_RELS, jnp.int32)]
  ).reshape(NW, NSUP, IDXC, CHUNK)
  gidx4 = jnp.concatenate([src_p, et_p + N], axis=3)
  dst4 = jnp.concatenate([dst, spread]).reshape(NW, NSUP, IDXC, CHUNK)

  re_ext = jnp.concatenate(
      [rel_embed, jnp.zeros((1, D), rel_embed.dtype)], axis=0)
  tab_bf = jnp.concatenate([x, re_ext], axis=0).astype(jnp.bfloat16)
  tab = jax.lax.bitcast_convert_type(
      tab_bf.reshape(N + 401, D // 2, 2), jnp.int32)
  zrows = jnp.zeros((ROWS_PT, D), jnp.float32)

  partials = _sc_message_accumulate(gidx4, dst4, tab, zrows)
  ar = jnp.arange(D)
  r32 = ar % 32
  perm = 32 * (ar // 32) + jnp.where(r32 < 16, 2 * r32, 2 * (r32 - 16) + 1)
  out = _tc_out_matmul(partials, weight[perm], bias.reshape(1, D))[:N]

  re_pad = jnp.concatenate(
      [re_ext, jnp.zeros((7, D), rel_embed.dtype)], axis=0)
  rel_out = _tc_rel_matmul(re_pad, rel_weight)[:2 * NUM_RELS + 1]
  return (out, rel_out)


# speedup vs baseline: 1.3001x; 1.0130x over previous
import functools

import jax
import jax.numpy as jnp
from jax import lax
from jax.experimental import pallas as pl
from jax.experimental.pallas import tpu as pltpu
from jax.experimental.pallas import tpu_sc as plsc

N = 10000
E = 320000
D = 128
NUM_RELS = 200

NC = 2
NS = 16
NW = NC * NS
CHUNK = 32
GIDX = 2 * CHUNK
NCHUNK = 320
EPW = NCHUNK * CHUNK
IDXC = 40
NSUP = NCHUNK // IDXC
RING = 5
LEAD = RING - 2
GRP = RING
NGRP = IDXC // GRP
EPAD = EPW * NW
NPAD = 10112
ROWS_PT = NPAD // NS
LANES = 16


def _sc_message_accumulate(gidx4, dst4, tab, zrows):
  mesh = plsc.VectorSubcoreMesh(core_axis_name="c", subcore_axis_name="s")

  @functools.partial(
      pl.kernel,
      out_type=jax.ShapeDtypeStruct((NC, NPAD, D), jnp.float32),
      mesh=mesh,
      compiler_params=pltpu.CompilerParams(use_tc_tiling_on_sc=False),
      scratch_types=[
          pltpu.VMEM((IDXC, GIDX), jnp.int32),
          pltpu.VMEM((IDXC, CHUNK), jnp.int32),
          [pltpu.VMEM((GIDX, D // 2), jnp.int32) for _ in range(RING)],
          [pltpu.VMEM((CHUNK, D), jnp.float32) for _ in range(RING)],
          pltpu.VMEM_SHARED((NPAD, D), jnp.float32),
          [pltpu.SemaphoreType.DMA for _ in range(RING)],
          [pltpu.SemaphoreType.DMA for _ in range(RING)],
      ],
  )
  def k(gidx_hbm, dst_hbm, tab_hbm, z_hbm, out_hbm,
        gidx_v, dst_v, bufs, prods, acc_sh, sem_g, sem_s):
    cid = lax.axis_index("c")
    sid = lax.axis_index("s")
    wid = cid * NS + sid

    pltpu.sync_copy(z_hbm, acc_sh.at[pl.ds(sid * ROWS_PT, ROWS_PT)])
    plsc.subcore_barrier()

    def issue_gather(c, r):
      pltpu.async_copy(tab_hbm.at[gidx_v.at[c]], bufs[r], sem_g[r])

    def wait_gather(c, r):
      pltpu.make_async_copy(tab_hbm.at[gidx_v.at[c]], bufs[r],
                            sem_g[r]).wait()

    def issue_scatter(c, r):
      pltpu.async_copy(prods[r], acc_sh.at[dst_v.at[c]], sem_s[r], add=True)

    def wait_scatter(c, r):
      pltpu.make_async_copy(prods[r], acc_sh.at[dst_v.at[c]],
                            sem_s[r]).wait()

    def mul(r):
      buf = bufs[r]
      prod = prods[r]

      hi_mask = jnp.int32(-65536)

      def mul_body(e, carry2):
        for j in range(D // (2 * LANES)):
          s = pl.ds(j * LANES, LANES)
          wx = buf[e, s]
          wr = buf[CHUNK + e, s]
          xa = lax.bitcast_convert_type(wx << 16, jnp.float32)
          xb = lax.bitcast_convert_type(wx & hi_mask, jnp.float32)
          ra = lax.bitcast_convert_type(wr << 16, jnp.float32)
          rb = lax.bitcast_convert_type(wr & hi_mask, jnp.float32)
          prod[e, pl.ds(j * 2 * LANES, LANES)] = xa * ra
          prod[e, pl.ds(j * 2 * LANES + LANES, LANES)] = xb * rb
        return carry2

      lax.fori_loop(0, CHUNK, mul_body, 0)

    def super_body(sc, carry):
      pltpu.sync_copy(gidx_hbm.at[wid, sc], gidx_v)
      pltpu.sync_copy(dst_hbm.at[wid, sc], dst_v)

      for c in range(LEAD):
        issue_gather(c, c % RING)

      def group_body(g, carry1):
        base = g * GRP
        for k in range(GRP):
          c = base + k
          if k >= LEAD:
            wait_scatter(c - LEAD, (k - LEAD) % RING)
          else:
            @pl.when(g >= 1)
            def _():
              wait_scatter(c - LEAD, (k - LEAD + GRP) % RING)
          if k < GRP - LEAD:
            issue_gather(c + LEAD, (k + LEAD) % RING)
          else:
            @pl.when(g + 1 < NGRP)
            def _():
              issue_gather(c + LEAD, (k + LEAD) % RING)
          wait_gather(c, k % RING)
          mul(k % RING)
          issue_scatter(c, k % RING)
        return carry1

      lax.fori_loop(0, NGRP, group_body, 0)
      for i in range(LEAD):
        c = IDXC - LEAD + i
        wait_scatter(c, c % RING)
      return carry

    lax.fori_loop(0, NSUP, super_body, 0)
    plsc.subcore_barrier()

    r0 = sid * ROWS_PT
    pltpu.sync_copy(acc_sh.at[pl.ds(r0, ROWS_PT)],
                    out_hbm.at[cid, pl.ds(r0, ROWS_PT)])

  return k(gidx4, dst4, tab, zrows)


def _tc_out_matmul(partials, weight, bias2d):
  BM = 1264

  def body(p_ref, w_ref, b_ref, o_ref):
    acc = jnp.dot(p_ref[0] + p_ref[1], w_ref[...],
                  preferred_element_type=jnp.float32)
    o_ref[...] = acc + b_ref[...]

  return pl.pallas_call(
      body,
      grid=(NPAD // BM,),
      in_specs=[
          pl.BlockSpec((NC, BM, D), lambda i: (0, i, 0)),
          pl.BlockSpec((D, D), lambda i: (0, 0)),
          pl.BlockSpec((1, D), lambda i: (0, 0)),
      ],
      out_specs=pl.BlockSpec((BM, D), lambda i: (i, 0)),
      out_shape=jax.ShapeDtypeStruct((NPAD, D), jnp.float32),
  )(partials, weight, bias2d)


def _tc_rel_matmul(re_pad, rel_weight):
  def body(r_ref, w_ref, o_ref):
    o_ref[...] = jnp.dot(r_ref[...], w_ref[...],
                         preferred_element_type=jnp.float32)

  return pl.pallas_call(
      body,
      out_shape=jax.ShapeDtypeStruct((re_pad.shape[0], D), jnp.float32),
  )(re_pad, rel_weight)


def kernel(x, edge_index, edge_type, rel_embed, weight, rel_weight, bias):
  src = edge_index[0]
  dst = edge_index[1]
  npad = EPAD - E
  spread = jnp.arange(npad, dtype=jnp.int32) % N
  src_p = jnp.concatenate([src, spread]).reshape(NW, NSUP, IDXC, CHUNK)
  et_p = jnp.concatenate(
      [edge_type, jnp.full((npad,), 2 * NUM_RELS, jnp.int32)]
  ).reshape(NW, NSUP, IDXC, CHUNK)
  gidx4 = jnp.concatenate([src_p, et_p + N], axis=3)
  dst4 = jnp.concatenate([dst, spread]).reshape(NW, NSUP, IDXC, CHUNK)

  re_ext = jnp.concatenate(
      [rel_embed, jnp.zeros((1, D), rel_embed.dtype)], axis=0)
  tab_bf = jnp.concatenate([x, re_ext], axis=0).astype(jnp.bfloat16)
  tab = jax.lax.bitcast_convert_type(
      tab_bf.reshape(N + 401, D // 2, 2), jnp.int32)
  zrows = jnp.zeros((ROWS_PT, D), jnp.float32)

  partials = _sc_message_accumulate(gidx4, dst4, tab, zrows)
  ar = jnp.arange(D)
  r32 = ar % 32
  perm = 32 * (ar // 32) + jnp.where(r32 < 16, 2 * r32, 2 * (r32 - 16) + 1)
  out = _tc_out_matmul(partials, weight[perm], bias.reshape(1, D))[:N]

  re_pad = jnp.concatenate(
      [re_ext, jnp.zeros((7, D), rel_embed.dtype)], axis=0)
  rel_out = _tc_rel_matmul(re_pad, rel_weight)[:2 * NUM_RELS + 1]
  return (out, rel_out)


# speedup vs baseline: 1.3660x; 1.0507x over previous
import functools

import jax
import jax.numpy as jnp
from jax import lax
from jax.experimental import pallas as pl
from jax.experimental.pallas import tpu as pltpu
from jax.experimental.pallas import tpu_sc as plsc

N = 10000
E = 320000
D = 128
NUM_RELS = 200

NC = 2
NS = 16
NW = NC * NS
CHUNK = 32
GIDX = 2 * CHUNK
NCHUNK = 320
EPW = NCHUNK * CHUNK
IDXC = 80
NSUP = NCHUNK // IDXC
RING = 5
LEAD = RING - 2
GRP = RING
NGRP = IDXC // GRP
EPAD = EPW * NW
NPAD = 10112
ROWS_PT = NPAD // NS
LANES = 16


def _sc_message_accumulate(gidx4, dst4, tab, zrows):
  mesh = plsc.VectorSubcoreMesh(core_axis_name="c", subcore_axis_name="s")

  @functools.partial(
      pl.kernel,
      out_type=jax.ShapeDtypeStruct((NC, NPAD, D), jnp.float32),
      mesh=mesh,
      compiler_params=pltpu.CompilerParams(use_tc_tiling_on_sc=False),
      scratch_types=[
          pltpu.VMEM((IDXC, GIDX), jnp.int32),
          pltpu.VMEM((IDXC, CHUNK), jnp.int32),
          [pltpu.VMEM((GIDX, D // 2), jnp.int32) for _ in range(RING)],
          [pltpu.VMEM((CHUNK, D), jnp.float32) for _ in range(RING)],
          pltpu.VMEM_SHARED((NPAD, D), jnp.float32),
          [pltpu.SemaphoreType.DMA for _ in range(RING)],
          [pltpu.SemaphoreType.DMA for _ in range(RING)],
      ],
  )
  def k(gidx_hbm, dst_hbm, tab_hbm, z_hbm, out_hbm,
        gidx_v, dst_v, bufs, prods, acc_sh, sem_g, sem_s):
    cid = lax.axis_index("c")
    sid = lax.axis_index("s")
    wid = cid * NS + sid

    pltpu.sync_copy(z_hbm, acc_sh.at[pl.ds(sid * ROWS_PT, ROWS_PT)])
    plsc.subcore_barrier()

    def issue_gather(c, r):
      pltpu.async_copy(tab_hbm.at[gidx_v.at[c]], bufs[r], sem_g[r])

    def wait_gather(c, r):
      pltpu.make_async_copy(tab_hbm.at[gidx_v.at[c]], bufs[r],
                            sem_g[r]).wait()

    def issue_scatter(c, r):
      pltpu.async_copy(prods[r], acc_sh.at[dst_v.at[c]], sem_s[r], add=True)

    def wait_scatter(c, r):
      pltpu.make_async_copy(prods[r], acc_sh.at[dst_v.at[c]],
                            sem_s[r]).wait()

    def mul(r):
      buf = bufs[r]
      prod = prods[r]

      hi_mask = jnp.int32(-65536)

      def mul_body(e, carry2):
        for j in range(D // (2 * LANES)):
          s = pl.ds(j * LANES, LANES)
          wx = buf[e, s]
          wr = buf[CHUNK + e, s]
          xa = lax.bitcast_convert_type(wx << 16, jnp.float32)
          xb = lax.bitcast_convert_type(wx & hi_mask, jnp.float32)
          ra = lax.bitcast_convert_type(wr << 16, jnp.float32)
          rb = lax.bitcast_convert_type(wr & hi_mask, jnp.float32)
          prod[e, pl.ds(j * 2 * LANES, LANES)] = xa * ra
          prod[e, pl.ds(j * 2 * LANES + LANES, LANES)] = xb * rb
        return carry2

      lax.fori_loop(0, CHUNK, mul_body, 0)

    def super_body(sc, carry):
      pltpu.sync_copy(gidx_hbm.at[wid, sc], gidx_v)
      pltpu.sync_copy(dst_hbm.at[wid, sc], dst_v)

      for c in range(LEAD):
        issue_gather(c, c % RING)

      def group_body(g, carry1):
        base = g * GRP
        for k in range(GRP):
          c = base + k
          if k >= LEAD:
            wait_scatter(c - LEAD, (k - LEAD) % RING)
          else:
            @pl.when(g >= 1)
            def _():
              wait_scatter(c - LEAD, (k - LEAD + GRP) % RING)
          if k < GRP - LEAD:
            issue_gather(c + LEAD, (k + LEAD) % RING)
          else:
            @pl.when(g + 1 < NGRP)
            def _():
              issue_gather(c + LEAD, (k + LEAD) % RING)
          wait_gather(c, k % RING)
          mul(k % RING)
          issue_scatter(c, k % RING)
        return carry1

      lax.fori_loop(0, NGRP, group_body, 0)
      for i in range(LEAD):
        c = IDXC - LEAD + i
        wait_scatter(c, c % RING)
      return carry

    lax.fori_loop(0, NSUP, super_body, 0)
    plsc.subcore_barrier()

    r0 = sid * ROWS_PT
    pltpu.sync_copy(acc_sh.at[pl.ds(r0, ROWS_PT)],
                    out_hbm.at[cid, pl.ds(r0, ROWS_PT)])

  return k(gidx4, dst4, tab, zrows)


def _tc_out_matmul(partials, weight, bias2d):
  BM = 1264

  def body(p_ref, w_ref, b_ref, o_ref):
    acc = jnp.dot(p_ref[0] + p_ref[1], w_ref[...],
                  preferred_element_type=jnp.float32)
    o_ref[...] = acc + b_ref[...]

  return pl.pallas_call(
      body,
      grid=(NPAD // BM,),
      in_specs=[
          pl.BlockSpec((NC, BM, D), lambda i: (0, i, 0)),
          pl.BlockSpec((D, D), lambda i: (0, 0)),
          pl.BlockSpec((1, D), lambda i: (0, 0)),
      ],
      out_specs=pl.BlockSpec((BM, D), lambda i: (i, 0)),
      out_shape=jax.ShapeDtypeStruct((NPAD, D), jnp.float32),
  )(partials, weight, bias2d)


def _tc_rel_matmul(re_pad, rel_weight):
  def body(r_ref, w_ref, o_ref):
    o_ref[...] = jnp.dot(r_ref[...], w_ref[...],
                         preferred_element_type=jnp.float32)

  return pl.pallas_call(
      body,
      out_shape=jax.ShapeDtypeStruct((re_pad.shape[0], D), jnp.float32),
  )(re_pad, rel_weight)


def kernel(x, edge_index, edge_type, rel_embed, weight, rel_weight, bias):
  src = edge_index[0]
  dst = edge_index[1]
  npad = EPAD - E
  spread = jnp.arange(npad, dtype=jnp.int32) % N
  src_p = jnp.concatenate([src, spread]).reshape(NW, NSUP, IDXC, CHUNK)
  et_p = jnp.concatenate(
      [edge_type, jnp.full((npad,), 2 * NUM_RELS, jnp.int32)]
  ).reshape(NW, NSUP, IDXC, CHUNK)
  gidx4 = jnp.concatenate([src_p, et_p + N], axis=3)
  dst4 = jnp.concatenate([dst, spread]).reshape(NW, NSUP, IDXC, CHUNK)

  re_ext = jnp.concatenate(
      [rel_embed, jnp.zeros((1, D), rel_embed.dtype)], axis=0)
  tab_bf = jnp.concatenate([x, re_ext], axis=0).astype(jnp.bfloat16)
  tab = jax.lax.bitcast_convert_type(
      tab_bf.reshape(N + 401, D // 2, 2), jnp.int32)
  zrows = jnp.zeros((ROWS_PT, D), jnp.float32)

  partials = _sc_message_accumulate(gidx4, dst4, tab, zrows)
  ar = jnp.arange(D)
  r32 = ar % 32
  perm = 32 * (ar // 32) + jnp.where(r32 < 16, 2 * r32, 2 * (r32 - 16) + 1)
  out = _tc_out_matmul(partials, weight[perm], bias.reshape(1, D))[:N]

  re_pad = jnp.concatenate(
      [re_ext, jnp.zeros((7, D), rel_embed.dtype)], axis=0)
  rel_out = _tc_rel_matmul(re_pad, rel_weight)[:2 * NUM_RELS + 1]
  return (out, rel_out)
